# baseline (speedup 1.0000x reference)
import functools

import jax
import jax.numpy as jnp
from jax import lax
from jax.experimental import pallas as pl
from jax.experimental.pallas import tpu as pltpu
from jax.experimental.pallas import tpu_sc as plsc

N = 50000
E = 800000
NT = 32
NO = 8
NQ = 4
ES = E // NQ
C = 2000
NCHUNK = ES // C
NPAIR = NCHUNK // 2
E_PAD = 819200
EB = 8192

SEGC = 12544
SEGD = 1664

_SC_PARAMS = pltpu.CompilerParams(needs_layout_passes=False)


def _elu16(h):
    return jnp.where(h > 0, h, jnp.exp(jnp.minimum(h, 0.0)) - 1.0)


def _phase_a_body(pt_ref, mt_ref, *bt_refs):
    pt = pt_ref[...]
    v = pt * 4.0
    io8 = lax.broadcasted_iota(jnp.int32, (8, EB), 0).astype(jnp.float32)

    def cdim(d):
        return jnp.maximum(1.0 - jnp.abs(io8 - v[d:d + 1]), 0.0)

    c0 = cdim(0)
    c1 = cdim(1)
    c2 = cdim(2)
    c12 = (c2[:, None, :] * c1[None, :, :]).reshape(64, EB)
    dt = jax.lax.dot_general(mt_ref[...], c12, (((1,), (0,)), ((), ())),
                             preferred_element_type=jnp.float32)
    acc = dt[0:8] * c0[0:1]
    for i in range(1, 5):
        acc = acc + dt[i * 8:(i + 1) * 8] * c0[i:i + 1]
    for r in range(8):
        bt_refs[r][...] = acc[r]


def _phase_a(pseudo_t, mth):
    return pl.pallas_call(
        _phase_a_body,
        grid=(E_PAD // EB,),
        in_specs=[pl.BlockSpec((3, EB), lambda i: (0, i)),
                  pl.BlockSpec((40, 64), lambda i: (0, 0))],
        out_specs=[pl.BlockSpec((EB,), lambda i: (i,)) for _ in range(8)],
        out_shape=[jax.ShapeDtypeStruct((E_PAD,), jnp.float32)
                   for _ in range(8)],
    )(pseudo_t, mth)


def _make_sc_layer(table_rows):
    mesh = plsc.VectorSubcoreMesh(core_axis_name="c", subcore_axis_name="s")

    @functools.partial(
        pl.kernel,
        mesh=mesh,
        compiler_params=_SC_PARAMS,
        out_type=jax.ShapeDtypeStruct((NT * N,), jnp.float32),
        scratch_types=[
            pltpu.VMEM((N,), jnp.float32),
            pltpu.VMEM((N,), jnp.float32),
            pltpu.VMEM((C,), jnp.int32),
            pltpu.VMEM((C,), jnp.int32),
            pltpu.VMEM((C,), jnp.float32),
            pltpu.VMEM((C,), jnp.int32),
            pltpu.VMEM((C,), jnp.int32),
            pltpu.VMEM((C,), jnp.float32),
            pltpu.SemaphoreType.DMA,
            pltpu.SemaphoreType.DMA,
        ],
    )
    def sc_layer(table_hbm, src_hbm, dst_hbm, b0, b1, b2, b3, b4, b5, b6, b7,
                 out_hbm, tab_v, agg_v, sA, dA, bA, sB, dB, bB, semA, semB):
        brows = (b0, b1, b2, b3, b4, b5, b6, b7)
        wid = lax.axis_index("s") * 2 + lax.axis_index("c")
        o = wid % NO
        q = wid // NO
        if table_rows == NO:
            pltpu.sync_copy(table_hbm.at[pl.ds(o * N, N)], tab_v)
        else:
            pltpu.sync_copy(table_hbm, tab_v)
        zeros16 = jnp.zeros((16,), jnp.float32)

        @plsc.parallel_loop(0, N, step=16, unroll=8)
        def _zero(i):
            agg_v[pl.ds(i, 16)] = zeros16

        ebase = q * ES

        def fire(sv, dv, bv, sem, off):
            pltpu.async_copy(src_hbm.at[pl.ds(off, C)], sv, sem)
            pltpu.async_copy(dst_hbm.at[pl.ds(off, C)], dv, sem)
            for r in range(NO):
                @pl.when(o == r)
                def _(_r=r):
                    pltpu.async_copy(brows[_r].at[pl.ds(off, C)], bv, sem)

        def wait(sv, dv, bv, sem):
            pltpu.make_async_copy(src_hbm.at[pl.ds(0, C)], sv, sem).wait()
            pltpu.make_async_copy(dst_hbm.at[pl.ds(0, C)], dv, sem).wait()
            pltpu.make_async_copy(b0.at[pl.ds(0, C)], bv, sem).wait()

        def compute(sv, dv, bv):
            @plsc.parallel_loop(0, C, step=16, unroll=5)
            def _inner(j):
                sl = pl.ds(j, 16)
                si = sv[sl]
                xs = plsc.load_gather(tab_v, [si])
                msg = xs * bv[sl]
                di = dv[sl]
                plsc.addupdate_scatter(agg_v, [di], msg)

        fire(sA, dA, bA, semA, pl.multiple_of(ebase, 8))

        def pair(k, carry):
            fire(sB, dB, bB, semB,
                 pl.multiple_of(ebase + (2 * k + 1) * C, 8))
            wait(sA, dA, bA, semA)
            compute(sA, dA, bA)

            @pl.when(k < NPAIR - 1)
            def _():
                fire(sA, dA, bA, semA,
                     pl.multiple_of(ebase + (2 * k + 2) * C, 8))

            wait(sB, dB, bB, semB)
            compute(sB, dB, bB)
            return carry

        lax.fori_loop(0, NPAIR, pair, 0)
        pltpu.sync_copy(agg_v, out_hbm.at[pl.ds(wid * N, N)])

    return sc_layer


_sc_layer1 = _make_sc_layer(1)
_sc_layer2 = _make_sc_layer(NO)


def _make_sc_combine1():
    mesh = plsc.VectorSubcoreMesh(core_axis_name="c", subcore_axis_name="s")

    @functools.partial(
        pl.kernel,
        mesh=mesh,
        compiler_params=_SC_PARAMS,
        out_type=jax.ShapeDtypeStruct((NO * N,), jnp.float32),
        scratch_types=[
            pltpu.VMEM(((NQ + 1) * SEGC,), jnp.float32),
            pltpu.VMEM((SEGC,), jnp.float32),
            pltpu.VMEM((16,), jnp.float32),
            pltpu.VMEM((16,), jnp.float32),
            pltpu.SemaphoreType.DMA,
        ],
    )
    def sc_c(p_hbm, x_hbm, rb_hbm, bb_hbm, out_hbm, buf_v, out_v, r_v, b_v,
             sem):
        wid = lax.axis_index("s") * 2 + lax.axis_index("c")
        o = wid % NO
        q = wid // NO
        start = pl.multiple_of(
            jnp.minimum(q * SEGC, N - SEGC).astype(jnp.int32), 16)
        pltpu.sync_copy(rb_hbm.at[pl.ds(o * 16, 16)], r_v)
        pltpu.sync_copy(bb_hbm.at[pl.ds(o * 16, 16)], b_v)
        cps = []
        for j in range(NQ):
            cps.append(pltpu.async_copy(
                p_hbm.at[pl.ds((j * NO + o) * N + start, SEGC)],
                buf_v.at[pl.ds(j * SEGC, SEGC)], sem))
        cps.append(pltpu.async_copy(x_hbm.at[pl.ds(start, SEGC)],
                                    buf_v.at[pl.ds(NQ * SEGC, SEGC)], sem))
        for cp in cps:
            cp.wait()
        rv = r_v[...]
        bv = b_v[...]

        def fbody(k, carry):
            s = buf_v[pl.ds(k * 16, 16)]
            for j in range(1, NQ):
                s = s + buf_v[pl.ds(j * SEGC + k * 16, 16)]
            h = s + buf_v[pl.ds(NQ * SEGC + k * 16, 16)] * rv + bv
            out_v[pl.ds(k * 16, 16)] = _elu16(h)
            return carry

        lax.fori_loop(0, SEGC // 16, fbody, 0, unroll=4)
        pltpu.sync_copy(out_v, out_hbm.at[pl.ds(o * N + start, SEGC)])

    return sc_c


_sc_combine1 = _make_sc_combine1()


def _make_sc_combine2():
    mesh = plsc.VectorSubcoreMesh(core_axis_name="c", subcore_axis_name="s")

    @functools.partial(
        pl.kernel,
        mesh=mesh,
        compiler_params=_SC_PARAMS,
        out_type=jax.ShapeDtypeStruct((N,), jnp.float32),
        scratch_types=[
            pltpu.VMEM(((NT + NO) * SEGD,), jnp.float32),
            pltpu.VMEM((SEGD,), jnp.float32),
            pltpu.VMEM((128,), jnp.float32),
            pltpu.VMEM((16,), jnp.float32),
            pltpu.SemaphoreType.DMA,
        ],
    )
    def sc_d(p_hbm, h_hbm, rb_hbm, bb_hbm, out_hbm, buf_v, acc_v, r_v, b_v,
             sem):
        wid = lax.axis_index("s") * 2 + lax.axis_index("c")
        start = pl.multiple_of(
            jnp.minimum(wid * SEGD, N - SEGD).astype(jnp.int32), 16)
        pltpu.sync_copy(rb_hbm, r_v)
        pltpu.sync_copy(bb_hbm, b_v)
        cps = []
        for j in range(NT):
            cps.append(pltpu.async_copy(
                p_hbm.at[pl.ds(j * N + start, SEGD)],
                buf_v.at[pl.ds(j * SEGD, SEGD)], sem))
        for o2 in range(NO):
            cps.append(pltpu.async_copy(
                h_hbm.at[pl.ds(o2 * N + start, SEGD)],
                buf_v.at[pl.ds((NT + o2) * SEGD, SEGD)], sem))
        for cp in cps:
            cp.wait()
        rv = [r_v[pl.ds(o2 * 16, 16)] for o2 in range(NO)]
        bv = b_v[...]

        def fbody(k, carry):
            s = buf_v[pl.ds(k * 16, 16)]
            for j in range(1, NT):
                s = s + buf_v[pl.ds(j * SEGD + k * 16, 16)]
            for o2 in range(NO):
                s = s + buf_v[pl.ds((NT + o2) * SEGD + k * 16, 16)] * rv[o2]
            acc_v[pl.ds(k * 16, 16)] = _elu16(s + bv)
            return carry

        lax.fori_loop(0, SEGD // 16, fbody, 0)
        pltpu.sync_copy(acc_v, out_hbm.at[pl.ds(start, SEGD)])

    return sc_d


_sc_combine2 = _make_sc_combine2()


def kernel(x, edge_index, pseudo, W1, root1, bias1, W2, root2, bias2):
    src = edge_index[0].astype(jnp.int32)
    dst = edge_index[1].astype(jnp.int32)
    pseudo_t = jnp.pad(pseudo.T, ((0, 0), (0, E_PAD - E)))
    wcat = jnp.concatenate([W1[:, 0, :], W2[:, :, 0]], axis=1)
    mt25 = wcat.reshape(5, 5, 5, 16).reshape(25, 80).T
    cols = jnp.array([8 * (m // 5) + m % 5 for m in range(25)], jnp.int32)
    mt = jnp.zeros((80, 64), jnp.float32).at[:, cols].set(mt25)
    mtr = mt.reshape(5, 16, 64)
    mt1 = mtr[:, 0:8].reshape(40, 64)
    mt2 = mtr[:, 8:16].reshape(40, 64)
    xf = x.reshape(-1)
    rb1 = jnp.tile(root1.reshape(NO, 1), (1, 16)).reshape(-1)
    bb1 = jnp.tile(bias1.reshape(NO, 1), (1, 16)).reshape(-1)
    rb2 = jnp.tile(root2.reshape(NO, 1), (1, 16)).reshape(-1)
    bb2 = jnp.broadcast_to(bias2, (16,)).astype(jnp.float32)

    bt1 = _phase_a(pseudo_t, mt1)
    bt2 = _phase_a(pseudo_t, mt2)
    p1 = _sc_layer1(xf, src, dst, *bt1)
    htf = _sc_combine1(p1, xf, rb1, bb1)
    p2 = _sc_layer2(htf, src, dst, *bt2)
    return _sc_combine2(p2, htf, rb2, bb2)


# speedup vs baseline: 1.1808x; 1.1808x over previous
import functools

import jax
import jax.numpy as jnp
from jax import lax
from jax.experimental import pallas as pl
from jax.experimental.pallas import tpu as pltpu
from jax.experimental.pallas import tpu_sc as plsc

N = 50000
E = 800000
NT = 32
NO = 8
NQ = 4
ES = E // NQ
C = 2000
NCHUNK = ES // C
NPAIR = NCHUNK // 2
E_PAD = 819200
EB = 8192

SEGC = 12544
SEGD = 1664

_SC_PARAMS = pltpu.CompilerParams(needs_layout_passes=False)


def _elu16(h):
    return jnp.where(h > 0, h, jnp.exp(jnp.minimum(h, 0.0)) - 1.0)


def _phase_a_body(pt_ref, mt_ref, *bt_refs):
    pt = pt_ref[...]
    v = pt * 4.0
    io8 = lax.broadcasted_iota(jnp.int32, (8, EB), 0).astype(jnp.float32)

    def cdim(d):
        return jnp.maximum(1.0 - jnp.abs(io8 - v[d:d + 1]), 0.0)

    c0 = cdim(0)
    c1 = cdim(1)
    c2 = cdim(2)
    c12 = (c2[:, None, :] * c1[None, :, :]).reshape(64, EB)
    dt = jax.lax.dot_general(mt_ref[...], c12, (((1,), (0,)), ((), ())),
                             preferred_element_type=jnp.float32)
    acc = dt[0:16] * c0[0:1]
    for i in range(1, 5):
        acc = acc + dt[i * 16:(i + 1) * 16] * c0[i:i + 1]
    for r in range(16):
        bt_refs[r][...] = acc[r]


def _phase_a(pseudo_t, mth):
    return pl.pallas_call(
        _phase_a_body,
        grid=(E_PAD // EB,),
        in_specs=[pl.BlockSpec((3, EB), lambda i: (0, i)),
                  pl.BlockSpec((80, 64), lambda i: (0, 0))],
        out_specs=[pl.BlockSpec((EB,), lambda i: (i,)) for _ in range(16)],
        out_shape=[jax.ShapeDtypeStruct((E_PAD,), jnp.float32)
                   for _ in range(16)],
    )(pseudo_t, mth)


def _make_sc_layer(table_rows):
    mesh = plsc.VectorSubcoreMesh(core_axis_name="c", subcore_axis_name="s")

    @functools.partial(
        pl.kernel,
        mesh=mesh,
        compiler_params=_SC_PARAMS,
        out_type=jax.ShapeDtypeStruct((NT * N,), jnp.float32),
        scratch_types=[
            pltpu.VMEM((N,), jnp.float32),
            pltpu.VMEM((N,), jnp.float32),
            pltpu.VMEM((C,), jnp.int32),
            pltpu.VMEM((C,), jnp.int32),
            pltpu.VMEM((C,), jnp.float32),
            pltpu.VMEM((C,), jnp.int32),
            pltpu.VMEM((C,), jnp.int32),
            pltpu.VMEM((C,), jnp.float32),
            pltpu.SemaphoreType.DMA,
            pltpu.SemaphoreType.DMA,
        ],
    )
    def sc_layer(table_hbm, src_hbm, dst_hbm, b0, b1, b2, b3, b4, b5, b6, b7,
                 out_hbm, tab_v, agg_v, sA, dA, bA, sB, dB, bB, semA, semB):
        brows = (b0, b1, b2, b3, b4, b5, b6, b7)
        wid = lax.axis_index("s") * 2 + lax.axis_index("c")
        o = wid % NO
        q = wid // NO
        if table_rows == NO:
            pltpu.sync_copy(table_hbm.at[pl.ds(o * N, N)], tab_v)
        else:
            pltpu.sync_copy(table_hbm, tab_v)
        zeros16 = jnp.zeros((16,), jnp.float32)

        @plsc.parallel_loop(0, N, step=16, unroll=8)
        def _zero(i):
            agg_v[pl.ds(i, 16)] = zeros16

        ebase = q * ES

        def fire(sv, dv, bv, sem, off):
            pltpu.async_copy(src_hbm.at[pl.ds(off, C)], sv, sem)
            pltpu.async_copy(dst_hbm.at[pl.ds(off, C)], dv, sem)
            for r in range(NO):
                @pl.when(o == r)
                def _(_r=r):
                    pltpu.async_copy(brows[_r].at[pl.ds(off, C)], bv, sem)

        def wait(sv, dv, bv, sem):
            pltpu.make_async_copy(src_hbm.at[pl.ds(0, C)], sv, sem).wait()
            pltpu.make_async_copy(dst_hbm.at[pl.ds(0, C)], dv, sem).wait()
            pltpu.make_async_copy(b0.at[pl.ds(0, C)], bv, sem).wait()

        def compute(sv, dv, bv):
            @plsc.parallel_loop(0, C, step=16, unroll=5)
            def _inner(j):
                sl = pl.ds(j, 16)
                si = sv[sl]
                xs = plsc.load_gather(tab_v, [si])
                msg = xs * bv[sl]
                di = dv[sl]
                plsc.addupdate_scatter(agg_v, [di], msg)

        fire(sA, dA, bA, semA, pl.multiple_of(ebase, 8))

        def pair(k, carry):
            fire(sB, dB, bB, semB,
                 pl.multiple_of(ebase + (2 * k + 1) * C, 8))
            wait(sA, dA, bA, semA)
            compute(sA, dA, bA)

            @pl.when(k < NPAIR - 1)
            def _():
                fire(sA, dA, bA, semA,
                     pl.multiple_of(ebase + (2 * k + 2) * C, 8))

            wait(sB, dB, bB, semB)
            compute(sB, dB, bB)
            return carry

        lax.fori_loop(0, NPAIR, pair, 0)
        pltpu.sync_copy(agg_v, out_hbm.at[pl.ds(wid * N, N)])

    return sc_layer


_sc_layer1 = _make_sc_layer(1)
_sc_layer2 = _make_sc_layer(NO)


def _make_sc_combine1():
    mesh = plsc.VectorSubcoreMesh(core_axis_name="c", subcore_axis_name="s")

    @functools.partial(
        pl.kernel,
        mesh=mesh,
        compiler_params=_SC_PARAMS,
        out_type=jax.ShapeDtypeStruct((NO * N,), jnp.float32),
        scratch_types=[
            pltpu.VMEM(((NQ + 1) * SEGC,), jnp.float32),
            pltpu.VMEM((SEGC,), jnp.float32),
            pltpu.VMEM((16,), jnp.float32),
            pltpu.VMEM((16,), jnp.float32),
            pltpu.SemaphoreType.DMA,
        ],
    )
    def sc_c(p_hbm, x_hbm, rb_hbm, bb_hbm, out_hbm, buf_v, out_v, r_v, b_v,
             sem):
        wid = lax.axis_index("s") * 2 + lax.axis_index("c")
        o = wid % NO
        q = wid // NO
        start = pl.multiple_of(
            jnp.minimum(q * SEGC, N - SEGC).astype(jnp.int32), 16)
        pltpu.sync_copy(rb_hbm.at[pl.ds(o * 16, 16)], r_v)
        pltpu.sync_copy(bb_hbm.at[pl.ds(o * 16, 16)], b_v)
        cps = []
        for j in range(NQ):
            cps.append(pltpu.async_copy(
                p_hbm.at[pl.ds((j * NO + o) * N + start, SEGC)],
                buf_v.at[pl.ds(j * SEGC, SEGC)], sem))
        cps.append(pltpu.async_copy(x_hbm.at[pl.ds(start, SEGC)],
                                    buf_v.at[pl.ds(NQ * SEGC, SEGC)], sem))
        for cp in cps:
            cp.wait()
        rv = r_v[...]
        bv = b_v[...]

        def fbody(k, carry):
            s = buf_v[pl.ds(k * 16, 16)]
            for j in range(1, NQ):
                s = s + buf_v[pl.ds(j * SEGC + k * 16, 16)]
            h = s + buf_v[pl.ds(NQ * SEGC + k * 16, 16)] * rv + bv
            out_v[pl.ds(k * 16, 16)] = _elu16(h)
            return carry

        lax.fori_loop(0, SEGC // 16, fbody, 0, unroll=4)
        pltpu.sync_copy(out_v, out_hbm.at[pl.ds(o * N + start, SEGC)])

    return sc_c


_sc_combine1 = _make_sc_combine1()


def _make_sc_combine2():
    mesh = plsc.VectorSubcoreMesh(core_axis_name="c", subcore_axis_name="s")

    @functools.partial(
        pl.kernel,
        mesh=mesh,
        compiler_params=_SC_PARAMS,
        out_type=jax.ShapeDtypeStruct((N,), jnp.float32),
        scratch_types=[
            pltpu.VMEM(((NT + NO) * SEGD,), jnp.float32),
            pltpu.VMEM((SEGD,), jnp.float32),
            pltpu.VMEM((128,), jnp.float32),
            pltpu.VMEM((16,), jnp.float32),
            pltpu.SemaphoreType.DMA,
        ],
    )
    def sc_d(p_hbm, h_hbm, rb_hbm, bb_hbm, out_hbm, buf_v, acc_v, r_v, b_v,
             sem):
        wid = lax.axis_index("s") * 2 + lax.axis_index("c")
        start = pl.multiple_of(
            jnp.minimum(wid * SEGD, N - SEGD).astype(jnp.int32), 16)
        pltpu.sync_copy(rb_hbm, r_v)
        pltpu.sync_copy(bb_hbm, b_v)
        cps = []
        for j in range(NT):
            cps.append(pltpu.async_copy(
                p_hbm.at[pl.ds(j * N + start, SEGD)],
                buf_v.at[pl.ds(j * SEGD, SEGD)], sem))
        for o2 in range(NO):
            cps.append(pltpu.async_copy(
                h_hbm.at[pl.ds(o2 * N + start, SEGD)],
                buf_v.at[pl.ds((NT + o2) * SEGD, SEGD)], sem))
        for cp in cps:
            cp.wait()
        rv = [r_v[pl.ds(o2 * 16, 16)] for o2 in range(NO)]
        bv = b_v[...]

        def fbody(k, carry):
            s = buf_v[pl.ds(k * 16, 16)]
            for j in range(1, NT):
                s = s + buf_v[pl.ds(j * SEGD + k * 16, 16)]
            for o2 in range(NO):
                s = s + buf_v[pl.ds((NT + o2) * SEGD + k * 16, 16)] * rv[o2]
            acc_v[pl.ds(k * 16, 16)] = _elu16(s + bv)
            return carry

        lax.fori_loop(0, SEGD // 16, fbody, 0)
        pltpu.sync_copy(acc_v, out_hbm.at[pl.ds(start, SEGD)])

    return sc_d


_sc_combine2 = _make_sc_combine2()


def kernel(x, edge_index, pseudo, W1, root1, bias1, W2, root2, bias2):
    src = edge_index[0].astype(jnp.int32)
    dst = edge_index[1].astype(jnp.int32)
    pseudo_t = jnp.pad(pseudo.T, ((0, 0), (0, E_PAD - E)))
    wcat = jnp.concatenate([W1[:, 0, :], W2[:, :, 0]], axis=1)
    mt25 = wcat.reshape(5, 5, 5, 16).reshape(25, 80).T
    cols = jnp.array([8 * (m // 5) + m % 5 for m in range(25)], jnp.int32)
    mt = jnp.zeros((80, 64), jnp.float32).at[:, cols].set(mt25)
    xf = x.reshape(-1)
    rb1 = jnp.tile(root1.reshape(NO, 1), (1, 16)).reshape(-1)
    bb1 = jnp.tile(bias1.reshape(NO, 1), (1, 16)).reshape(-1)
    rb2 = jnp.tile(root2.reshape(NO, 1), (1, 16)).reshape(-1)
    bb2 = jnp.broadcast_to(bias2, (16,)).astype(jnp.float32)

    bt = _phase_a(pseudo_t, mt)
    p1 = _sc_layer1(xf, src, dst, *bt[0:8])
    htf = _sc_combine1(p1, xf, rb1, bb1)
    p2 = _sc_layer2(htf, src, dst, *bt[8:16])
    return _sc_combine2(p2, htf, rb2, bb2)


# speedup vs baseline: 1.2847x; 1.0880x over previous
import functools

import jax
import jax.numpy as jnp
from jax import lax
from jax.experimental import pallas as pl
from jax.experimental.pallas import tpu as pltpu
from jax.experimental.pallas import tpu_sc as plsc

N = 50000
E = 800000
NT = 32
NO = 8
NQ = 4
ES = E // NQ
C = 4000
NCHUNK = ES // C
NPAIR = NCHUNK // 2
E_PAD = 819200
EB = 8192

SEGC = 12544
SEGD = 1664

_SC_PARAMS = pltpu.CompilerParams(needs_layout_passes=False)


def _elu16(h):
    return jnp.where(h > 0, h, jnp.exp(jnp.minimum(h, 0.0)) - 1.0)


def _phase_a_body(pt_ref, mt_ref, *bt_refs):
    pt = pt_ref[...]
    v = pt * 4.0
    io8 = lax.broadcasted_iota(jnp.int32, (8, EB), 0).astype(jnp.float32)

    def cdim(d):
        return jnp.maximum(1.0 - jnp.abs(io8 - v[d:d + 1]), 0.0)

    c0 = cdim(0)
    c1 = cdim(1)
    c2 = cdim(2)
    c12 = (c2[:, None, :] * c1[None, :, :]).reshape(64, EB)
    dt = jax.lax.dot_general(mt_ref[...], c12, (((1,), (0,)), ((), ())),
                             preferred_element_type=jnp.float32)
    acc = dt[0:16] * c0[0:1]
    for i in range(1, 5):
        acc = acc + dt[i * 16:(i + 1) * 16] * c0[i:i + 1]
    for r in range(16):
        bt_refs[r][...] = acc[r]


def _phase_a(pseudo_t, mth):
    return pl.pallas_call(
        _phase_a_body,
        grid=(E_PAD // EB,),
        in_specs=[pl.BlockSpec((3, EB), lambda i: (0, i)),
                  pl.BlockSpec((80, 64), lambda i: (0, 0))],
        out_specs=[pl.BlockSpec((EB,), lambda i: (i,)) for _ in range(16)],
        out_shape=[jax.ShapeDtypeStruct((E_PAD,), jnp.float32)
                   for _ in range(16)],
    )(pseudo_t, mth)


def _make_sc_layer(table_rows):
    mesh = plsc.VectorSubcoreMesh(core_axis_name="c", subcore_axis_name="s")

    @functools.partial(
        pl.kernel,
        mesh=mesh,
        compiler_params=_SC_PARAMS,
        out_type=jax.ShapeDtypeStruct((NT * N,), jnp.float32),
        scratch_types=[
            pltpu.VMEM((N,), jnp.float32),
            pltpu.VMEM((N,), jnp.float32),
            pltpu.VMEM((C,), jnp.int32),
            pltpu.VMEM((C,), jnp.int32),
            pltpu.VMEM((C,), jnp.float32),
            pltpu.VMEM((C,), jnp.int32),
            pltpu.VMEM((C,), jnp.int32),
            pltpu.VMEM((C,), jnp.float32),
            pltpu.SemaphoreType.DMA,
            pltpu.SemaphoreType.DMA,
        ],
    )
    def sc_layer(table_hbm, src_hbm, dst_hbm, b0, b1, b2, b3, b4, b5, b6, b7,
                 out_hbm, tab_v, agg_v, sA, dA, bA, sB, dB, bB, semA, semB):
        brows = (b0, b1, b2, b3, b4, b5, b6, b7)
        wid = lax.axis_index("s") * 2 + lax.axis_index("c")
        o = wid % NO
        q = wid // NO
        if table_rows == NO:
            pltpu.sync_copy(table_hbm.at[pl.ds(o * N, N)], tab_v)
        else:
            pltpu.sync_copy(table_hbm, tab_v)
        zeros16 = jnp.zeros((16,), jnp.float32)

        @plsc.parallel_loop(0, N, step=16, unroll=8)
        def _zero(i):
            agg_v[pl.ds(i, 16)] = zeros16

        ebase = q * ES

        def fire(sv, dv, bv, sem, off):
            pltpu.async_copy(src_hbm.at[pl.ds(off, C)], sv, sem)
            pltpu.async_copy(dst_hbm.at[pl.ds(off, C)], dv, sem)
            for r in range(NO):
                @pl.when(o == r)
                def _(_r=r):
                    pltpu.async_copy(brows[_r].at[pl.ds(off, C)], bv, sem)

        def wait(sv, dv, bv, sem):
            pltpu.make_async_copy(src_hbm.at[pl.ds(0, C)], sv, sem).wait()
            pltpu.make_async_copy(dst_hbm.at[pl.ds(0, C)], dv, sem).wait()
            pltpu.make_async_copy(b0.at[pl.ds(0, C)], bv, sem).wait()

        def compute(sv, dv, bv):
            @plsc.parallel_loop(0, C, step=16, unroll=5)
            def _inner(j):
                sl = pl.ds(j, 16)
                si = sv[sl]
                xs = plsc.load_gather(tab_v, [si])
                msg = xs * bv[sl]
                di = dv[sl]
                plsc.addupdate_scatter(agg_v, [di], msg)

        fire(sA, dA, bA, semA, pl.multiple_of(ebase, 8))

        def pair(k, carry):
            fire(sB, dB, bB, semB,
                 pl.multiple_of(ebase + (2 * k + 1) * C, 8))
            wait(sA, dA, bA, semA)
            compute(sA, dA, bA)

            @pl.when(k < NPAIR - 1)
            def _():
                fire(sA, dA, bA, semA,
                     pl.multiple_of(ebase + (2 * k + 2) * C, 8))

            wait(sB, dB, bB, semB)
            compute(sB, dB, bB)
            return carry

        lax.fori_loop(0, NPAIR, pair, 0)
        pltpu.sync_copy(agg_v, out_hbm.at[pl.ds(wid * N, N)])

    return sc_layer


_sc_layer1 = _make_sc_layer(1)
_sc_layer2 = _make_sc_layer(NO)


def _make_sc_combine1():
    mesh = plsc.VectorSubcoreMesh(core_axis_name="c", subcore_axis_name="s")

    @functools.partial(
        pl.kernel,
        mesh=mesh,
        compiler_params=_SC_PARAMS,
        out_type=jax.ShapeDtypeStruct((NO * N,), jnp.float32),
        scratch_types=[
            pltpu.VMEM(((NQ + 1) * SEGC,), jnp.float32),
            pltpu.VMEM((SEGC,), jnp.float32),
            pltpu.VMEM((16,), jnp.float32),
            pltpu.VMEM((16,), jnp.float32),
            pltpu.SemaphoreType.DMA,
        ],
    )
    def sc_c(p_hbm, x_hbm, rb_hbm, bb_hbm, out_hbm, buf_v, out_v, r_v, b_v,
             sem):
        wid = lax.axis_index("s") * 2 + lax.axis_index("c")
        o = wid % NO
        q = wid // NO
        start = pl.multiple_of(
            jnp.minimum(q * SEGC, N - SEGC).astype(jnp.int32), 16)
        pltpu.sync_copy(rb_hbm.at[pl.ds(o * 16, 16)], r_v)
        pltpu.sync_copy(bb_hbm.at[pl.ds(o * 16, 16)], b_v)
        cps = []
        for j in range(NQ):
            cps.append(pltpu.async_copy(
                p_hbm.at[pl.ds((j * NO + o) * N + start, SEGC)],
                buf_v.at[pl.ds(j * SEGC, SEGC)], sem))
        cps.append(pltpu.async_copy(x_hbm.at[pl.ds(start, SEGC)],
                                    buf_v.at[pl.ds(NQ * SEGC, SEGC)], sem))
        for cp in cps:
            cp.wait()
        rv = r_v[...]
        bv = b_v[...]

        def fbody(k, carry):
            s = buf_v[pl.ds(k * 16, 16)]
            for j in range(1, NQ):
                s = s + buf_v[pl.ds(j * SEGC + k * 16, 16)]
            h = s + buf_v[pl.ds(NQ * SEGC + k * 16, 16)] * rv + bv
            out_v[pl.ds(k * 16, 16)] = _elu16(h)
            return carry

        lax.fori_loop(0, SEGC // 16, fbody, 0, unroll=4)
        pltpu.sync_copy(out_v, out_hbm.at[pl.ds(o * N + start, SEGC)])

    return sc_c


_sc_combine1 = _make_sc_combine1()


def _make_sc_combine2():
    mesh = plsc.VectorSubcoreMesh(core_axis_name="c", subcore_axis_name="s")

    @functools.partial(
        pl.kernel,
        mesh=mesh,
        compiler_params=_SC_PARAMS,
        out_type=jax.ShapeDtypeStruct((N,), jnp.float32),
        scratch_types=[
            pltpu.VMEM(((NT + NO) * SEGD,), jnp.float32),
            pltpu.VMEM((SEGD,), jnp.float32),
            pltpu.VMEM((128,), jnp.float32),
            pltpu.VMEM((16,), jnp.float32),
            pltpu.SemaphoreType.DMA,
        ],
    )
    def sc_d(p_hbm, h_hbm, rb_hbm, bb_hbm, out_hbm, buf_v, acc_v, r_v, b_v,
             sem):
        wid = lax.axis_index("s") * 2 + lax.axis_index("c")
        start = pl.multiple_of(
            jnp.minimum(wid * SEGD, N - SEGD).astype(jnp.int32), 16)
        pltpu.sync_copy(rb_hbm, r_v)
        pltpu.sync_copy(bb_hbm, b_v)
        cps = []
        for j in range(NT):
            cps.append(pltpu.async_copy(
                p_hbm.at[pl.ds(j * N + start, SEGD)],
                buf_v.at[pl.ds(j * SEGD, SEGD)], sem))
        for o2 in range(NO):
            cps.append(pltpu.async_copy(
                h_hbm.at[pl.ds(o2 * N + start, SEGD)],
                buf_v.at[pl.ds((NT + o2) * SEGD, SEGD)], sem))
        for cp in cps:
            cp.wait()
        rv = [r_v[pl.ds(o2 * 16, 16)] for o2 in range(NO)]
        bv = b_v[...]

        def fbody(k, carry):
            s = buf_v[pl.ds(k * 16, 16)]
            for j in range(1, NT):
                s = s + buf_v[pl.ds(j * SEGD + k * 16, 16)]
            for o2 in range(NO):
                s = s + buf_v[pl.ds((NT + o2) * SEGD + k * 16, 16)] * rv[o2]
            acc_v[pl.ds(k * 16, 16)] = _elu16(s + bv)
            return carry

        lax.fori_loop(0, SEGD // 16, fbody, 0)
        pltpu.sync_copy(acc_v, out_hbm.at[pl.ds(start, SEGD)])

    return sc_d


_sc_combine2 = _make_sc_combine2()


def kernel(x, edge_index, pseudo, W1, root1, bias1, W2, root2, bias2):
    src = edge_index[0].astype(jnp.int32)
    dst = edge_index[1].astype(jnp.int32)
    pseudo_t = jnp.pad(pseudo.T, ((0, 0), (0, E_PAD - E)))
    wcat = jnp.concatenate([W1[:, 0, :], W2[:, :, 0]], axis=1)
    mt25 = wcat.reshape(5, 5, 5, 16).reshape(25, 80).T
    cols = jnp.array([8 * (m // 5) + m % 5 for m in range(25)], jnp.int32)
    mt = jnp.zeros((80, 64), jnp.float32).at[:, cols].set(mt25)
    xf = x.reshape(-1)
    rb1 = jnp.tile(root1.reshape(NO, 1), (1, 16)).reshape(-1)
    bb1 = jnp.tile(bias1.reshape(NO, 1), (1, 16)).reshape(-1)
    rb2 = jnp.tile(root2.reshape(NO, 1), (1, 16)).reshape(-1)
    bb2 = jnp.broadcast_to(bias2, (16,)).astype(jnp.float32)

    bt = _phase_a(pseudo_t, mt)
    p1 = _sc_layer1(xf, src, dst, *bt[0:8])
    htf = _sc_combine1(p1, xf, rb1, bb1)
    p2 = _sc_layer2(htf, src, dst, *bt[8:16])
    return _sc_combine2(p2, htf, rb2, bb2)
